# Initial kernel scaffold; baseline (speedup 1.0000x reference)
import jax
import jax.numpy as jnp
from jax.experimental import pallas as pl


def kernel(x, edge_index, batch, W1, b1, W2, b2, Wfc1, bfc1, Wout, bout):
    raise NotImplementedError("write your pallas kernel here")


# speedup vs baseline: 19.5526x; 19.5526x over previous
import functools

import jax
import jax.numpy as jnp
from jax import lax
from jax.experimental import pallas as pl
from jax.experimental.pallas import tpu as pltpu
from jax.experimental.pallas import tpu_sc as plsc

N = 10000
E = 320000
D = 128
H = 64
G = 64

NC = 2
NS = 16
NW = NC * NS
CH = 128
NCH = 80
EPT = CH * NCH
EPAD = EPT * NW
NP = 10240
RPT = NP // NS
HW = 16

@functools.cache
def _sc_mesh():
  return plsc.VectorSubcoreMesh(
      core_axis_name="c", subcore_axis_name="s", num_cores=NC, num_subcores=NS)


def _hist_body(dst_hbm, out_hbm, acc, onesv, idxv, bufv):
  c = lax.axis_index("c")
  s = lax.axis_index("s")

  def fill(i, carry):
    onesv[i, :] = jnp.full((HW,), 1.0, jnp.float32)
    bufv[i, :] = jnp.zeros((HW,), jnp.float32)
    return carry

  lax.fori_loop(0, CH, fill, 0)
  for t in range(RPT // CH):
    pltpu.sync_copy(bufv, acc.at[pl.ds(s * RPT + t * CH, CH)])
  plsc.subcore_barrier()
  pltpu.sync_copy(dst_hbm.at[c, s], idxv)

  def body(j, carry):
    pltpu.sync_copy(onesv, acc.at[idxv.at[j]], add=True)
    return carry

  lax.fori_loop(0, NCH, body, 0)
  plsc.subcore_barrier()
  for t in range(RPT // CH):
    pltpu.sync_copy(acc.at[pl.ds(s * RPT + t * CH, CH)], bufv)
    pltpu.sync_copy(bufv, out_hbm.at[c, pl.ds(s * RPT + t * CH, CH)])


@functools.cache
def _hist_call():
  return pl.kernel(
      _hist_body,
      out_type=jax.ShapeDtypeStruct((NC, NP, HW), jnp.float32),
      mesh=_sc_mesh(),
      compiler_params=pltpu.CompilerParams(use_tc_tiling_on_sc=False),
      scratch_types=[
          pltpu.VMEM_SHARED((NP, HW), jnp.float32),
          pltpu.VMEM((CH, HW), jnp.float32),
          pltpu.VMEM((NCH, CH), jnp.int32),
          pltpu.VMEM((CH, HW), jnp.float32),
      ],
  )


def _agg_body(src_hbm, dst_hbm, g_hbm, out_hbm, acc, srcv, dstv, rowsa, rowsb,
              sema, semb):
  c = lax.axis_index("c")
  s = lax.axis_index("s")

  def fill(i, carry):
    for q in range(H // 16):
      rowsa[i, pl.ds(q * 16, 16)] = jnp.zeros((16,), jnp.float32)
    return carry

  lax.fori_loop(0, CH, fill, 0)
  for t in range(RPT // CH):
    pltpu.sync_copy(rowsa, acc.at[pl.ds(s * RPT + t * CH, CH)])
  plsc.subcore_barrier()
  pltpu.sync_copy(src_hbm.at[c, s], srcv)
  pltpu.sync_copy(dst_hbm.at[c, s], dstv)
  pltpu.async_copy(g_hbm.at[srcv.at[0]], rowsa, sema)

  def body(t, carry):
    j = 2 * t
    db = pltpu.async_copy(g_hbm.at[srcv.at[j + 1]], rowsb, semb)
    pltpu.make_async_copy(g_hbm.at[srcv.at[0]], rowsa, sema).wait()
    pltpu.sync_copy(rowsa, acc.at[dstv.at[j]], add=True)

    @pl.when(t < NCH // 2 - 1)
    def _():
      pltpu.async_copy(g_hbm.at[srcv.at[j + 2]], rowsa, sema)

    db.wait()
    pltpu.sync_copy(rowsb, acc.at[dstv.at[j + 1]], add=True)
    return carry

  lax.fori_loop(0, NCH // 2, body, 0)
  plsc.subcore_barrier()
  for t in range(RPT // CH):
    pltpu.sync_copy(acc.at[pl.ds(s * RPT + t * CH, CH)], rowsa)
    pltpu.sync_copy(rowsa, out_hbm.at[c, pl.ds(s * RPT + t * CH, CH)])


@functools.cache
def _agg_call():
  return pl.kernel(
      _agg_body,
      out_type=jax.ShapeDtypeStruct((NC, NP, H), jnp.float32),
      mesh=_sc_mesh(),
      compiler_params=pltpu.CompilerParams(use_tc_tiling_on_sc=False),
      scratch_types=[
          pltpu.VMEM_SHARED((NP, H), jnp.float32),
          pltpu.VMEM((NCH, CH), jnp.int32),
          pltpu.VMEM((NCH, CH), jnp.int32),
          pltpu.VMEM((CH, H), jnp.float32),
          pltpu.VMEM((CH, H), jnp.float32),
          pltpu.SemaphoreType.DMA,
          pltpu.SemaphoreType.DMA,
      ],
  )


def _tc1_body(x_ref, w_ref, d0_ref, d1_ref, g_ref, dinv_ref):
  deg = d0_ref[...] + d1_ref[...] + 1.0
  dinv = lax.rsqrt(jnp.maximum(deg, 1.0))
  dinv_ref[...] = dinv
  g_ref[...] = dinv * jnp.dot(
      x_ref[...], w_ref[...], preferred_element_type=jnp.float32)


def _tc1_call(x, w, d0, d1):
  return pl.pallas_call(
      _tc1_body,
      out_shape=(
          jax.ShapeDtypeStruct((NP, H), jnp.float32),
          jax.ShapeDtypeStruct((NP, 1), jnp.float32),
      ),
  )(x, w, d0, d1)


def _tc2_body(p_ref, g_ref, dinv_ref, b_ref, w_ref, o_ref):
  es = p_ref[0] + p_ref[1] + g_ref[...]
  h = jnp.maximum(dinv_ref[...] * es + b_ref[...], 0.0)
  o_ref[...] = dinv_ref[...] * jnp.dot(
      h, w_ref[...], preferred_element_type=jnp.float32)


def _tc2_call(p, g, dinv, b, w):
  return pl.pallas_call(
      _tc2_body,
      out_shape=jax.ShapeDtypeStruct((NP, H), jnp.float32),
  )(p, g, dinv, b, w)


def _tc3_body(p_ref, g_ref, dinv_ref, b_ref, batch_ref, wfc_ref, bfc_ref,
              wout_ref, bout_ref, o_ref):
  h = jnp.maximum(
      dinv_ref[...] * (p_ref[0] + p_ref[1] + g_ref[...]) + b_ref[...], 0.0)
  seg = lax.broadcasted_iota(jnp.int32, (NP, G), 1)
  oh = (batch_ref[...] == seg).astype(jnp.float32)
  sums = lax.dot_general(
      oh, h, (((0,), (0,)), ((), ())), preferred_element_type=jnp.float32)
  counts = lax.dot_general(
      oh, jnp.ones((NP, 1), jnp.float32), (((0,), (0,)), ((), ())),
      preferred_element_type=jnp.float32)
  pooled = sums / jnp.maximum(counts, 1.0)
  z = jnp.maximum(
      jnp.dot(pooled, wfc_ref[...], preferred_element_type=jnp.float32)
      + bfc_ref[...], 0.0)
  o_ref[...] = jnp.dot(
      z, wout_ref[...], preferred_element_type=jnp.float32) + bout_ref[...]


def _tc3_call(p, g, dinv, b, batch2d, wfc, bfc, wout, bout):
  return pl.pallas_call(
      _tc3_body,
      out_shape=jax.ShapeDtypeStruct((G, 1), jnp.float32),
  )(p, g, dinv, b, batch2d, wfc, bfc, wout, bout)


def kernel(x, edge_index, batch, W1, b1, W2, b2, Wfc1, bfc1, Wout, bout):
  pad = jnp.full((EPAD - E,), N, jnp.int32)
  srcp = jnp.concatenate([edge_index[0].astype(jnp.int32), pad]).reshape(
      NC, NS, NCH, CH)
  dstp = jnp.concatenate([edge_index[1].astype(jnp.int32), pad]).reshape(
      NC, NS, NCH, CH)
  x_pad = jnp.pad(x, ((0, NP - N), (0, 0)))
  batch2d = jnp.pad(
      batch.astype(jnp.int32), (0, NP - N), constant_values=G).reshape(NP, 1)

  hist = _hist_call()(dstp)
  d0 = hist[0, :, 0:1]
  d1 = hist[1, :, 0:1]

  g1, dinv = _tc1_call(x_pad, W1, d0, d1)
  p1 = _agg_call()(srcp, dstp, g1)
  g2 = _tc2_call(p1, g1, dinv, b1.reshape(1, H), W2)
  p2 = _agg_call()(srcp, dstp, g2)
  out = _tc3_call(p2, g2, dinv, b2.reshape(1, H), batch2d, Wfc1,
                  bfc1.reshape(1, G), Wout, bout.reshape(1, 1))
  return out


# speedup vs baseline: 19.9430x; 1.0200x over previous
import functools

import jax
import jax.numpy as jnp
from jax import lax
from jax.experimental import pallas as pl
from jax.experimental.pallas import tpu as pltpu
from jax.experimental.pallas import tpu_sc as plsc

N = 10000
E = 320000
D = 128
H = 64
G = 64

NC = 2
NS = 16
NW = NC * NS
CH = 128
NCH = 80
EPT = CH * NCH
EPAD = EPT * NW
NP = 10240
RPT = NP // NS
HW = 16

@functools.cache
def _sc_mesh():
  return plsc.VectorSubcoreMesh(
      core_axis_name="c", subcore_axis_name="s", num_cores=NC, num_subcores=NS)


def _hist_body(dst_hbm, out_hbm, acc, onesv, idxv, bufv):
  c = lax.axis_index("c")
  s = lax.axis_index("s")

  def fill(i, carry):
    onesv[i, :] = jnp.full((HW,), 1.0, jnp.float32)
    bufv[i, :] = jnp.zeros((HW,), jnp.float32)
    return carry

  lax.fori_loop(0, CH, fill, 0)
  for t in range(RPT // CH):
    pltpu.sync_copy(bufv, acc.at[pl.ds(s * RPT + t * CH, CH)])
  plsc.subcore_barrier()
  pltpu.sync_copy(dst_hbm.at[c, s], idxv)

  def body(j, carry):
    pltpu.sync_copy(onesv, acc.at[idxv.at[j]], add=True)
    return carry

  lax.fori_loop(0, NCH, body, 0)
  plsc.subcore_barrier()
  for t in range(RPT // CH):
    pltpu.sync_copy(acc.at[pl.ds(s * RPT + t * CH, CH)], bufv)
    pltpu.sync_copy(bufv, out_hbm.at[c, pl.ds(s * RPT + t * CH, CH)])


@functools.cache
def _hist_call():
  return pl.kernel(
      _hist_body,
      out_type=jax.ShapeDtypeStruct((NC, NP, HW), jnp.float32),
      mesh=_sc_mesh(),
      compiler_params=pltpu.CompilerParams(use_tc_tiling_on_sc=False),
      scratch_types=[
          pltpu.VMEM_SHARED((NP, HW), jnp.float32),
          pltpu.VMEM((CH, HW), jnp.float32),
          pltpu.VMEM((NCH, CH), jnp.int32),
          pltpu.VMEM((CH, HW), jnp.float32),
      ],
  )


NBUF = 4


def _agg_body(src_hbm, dst_hbm, g_hbm, out_hbm, acc, srcv, dstv, rows, gsems,
              ssems):
  c = lax.axis_index("c")
  s = lax.axis_index("s")
  rowsa = rows[0]

  def fill(i, carry):
    for q in range(H // 16):
      rowsa[i, pl.ds(q * 16, 16)] = jnp.zeros((16,), jnp.float32)
    return carry

  lax.fori_loop(0, CH, fill, 0)
  for t in range(RPT // CH):
    pltpu.sync_copy(rowsa, acc.at[pl.ds(s * RPT + t * CH, CH)])
  plsc.subcore_barrier()
  pltpu.sync_copy(src_hbm.at[c, s], srcv)
  pltpu.sync_copy(dst_hbm.at[c, s], dstv)
  for b in range(NBUF):
    pltpu.async_copy(g_hbm.at[srcv.at[b]], rows[b], gsems[b])

  def body(t, carry):
    j0 = NBUF * t
    for b in range(NBUF):
      j = j0 + b
      pltpu.make_async_copy(g_hbm.at[srcv.at[0]], rows[b], gsems[b]).wait()
      pltpu.async_copy(rows[b], acc.at[dstv.at[j]], ssems[b], add=True)

      @pl.when(j + NBUF < NCH)
      def _():
        pltpu.make_async_copy(rows[b], acc.at[dstv.at[0]], ssems[b]).wait()
        pltpu.async_copy(g_hbm.at[srcv.at[j + NBUF]], rows[b], gsems[b])

    return carry

  lax.fori_loop(0, NCH // NBUF, body, 0)
  for b in range(NBUF):
    pltpu.make_async_copy(rows[b], acc.at[dstv.at[0]], ssems[b]).wait()
  plsc.subcore_barrier()
  for t in range(RPT // CH):
    pltpu.sync_copy(acc.at[pl.ds(s * RPT + t * CH, CH)], rowsa)
    pltpu.sync_copy(rowsa, out_hbm.at[c, pl.ds(s * RPT + t * CH, CH)])


@functools.cache
def _agg_call():
  return pl.kernel(
      _agg_body,
      out_type=jax.ShapeDtypeStruct((NC, NP, H), jnp.float32),
      mesh=_sc_mesh(),
      compiler_params=pltpu.CompilerParams(use_tc_tiling_on_sc=False),
      scratch_types=[
          pltpu.VMEM_SHARED((NP, H), jnp.float32),
          pltpu.VMEM((NCH, CH), jnp.int32),
          pltpu.VMEM((NCH, CH), jnp.int32),
          [pltpu.VMEM((CH, H), jnp.float32) for _ in range(NBUF)],
          [pltpu.SemaphoreType.DMA for _ in range(NBUF)],
          [pltpu.SemaphoreType.DMA for _ in range(NBUF)],
      ],
  )


def _tc1_body(x_ref, w_ref, d0_ref, d1_ref, g_ref, dinv_ref):
  deg = d0_ref[...] + d1_ref[...] + 1.0
  dinv = lax.rsqrt(jnp.maximum(deg, 1.0))
  dinv_ref[...] = dinv
  g_ref[...] = dinv * jnp.dot(
      x_ref[...], w_ref[...], preferred_element_type=jnp.float32)


def _tc1_call(x, w, d0, d1):
  return pl.pallas_call(
      _tc1_body,
      out_shape=(
          jax.ShapeDtypeStruct((NP, H), jnp.float32),
          jax.ShapeDtypeStruct((NP, 1), jnp.float32),
      ),
  )(x, w, d0, d1)


def _tc2_body(p_ref, g_ref, dinv_ref, b_ref, w_ref, o_ref):
  es = p_ref[0] + p_ref[1] + g_ref[...]
  h = jnp.maximum(dinv_ref[...] * es + b_ref[...], 0.0)
  o_ref[...] = dinv_ref[...] * jnp.dot(
      h, w_ref[...], preferred_element_type=jnp.float32)


def _tc2_call(p, g, dinv, b, w):
  return pl.pallas_call(
      _tc2_body,
      out_shape=jax.ShapeDtypeStruct((NP, H), jnp.float32),
  )(p, g, dinv, b, w)


def _tc3_body(p_ref, g_ref, dinv_ref, b_ref, batch_ref, wfc_ref, bfc_ref,
              wout_ref, bout_ref, o_ref):
  h = jnp.maximum(
      dinv_ref[...] * (p_ref[0] + p_ref[1] + g_ref[...]) + b_ref[...], 0.0)
  seg = lax.broadcasted_iota(jnp.int32, (NP, G), 1)
  oh = (batch_ref[...] == seg).astype(jnp.float32)
  sums = lax.dot_general(
      oh, h, (((0,), (0,)), ((), ())), preferred_element_type=jnp.float32)
  counts = lax.dot_general(
      oh, jnp.ones((NP, 1), jnp.float32), (((0,), (0,)), ((), ())),
      preferred_element_type=jnp.float32)
  pooled = sums / jnp.maximum(counts, 1.0)
  z = jnp.maximum(
      jnp.dot(pooled, wfc_ref[...], preferred_element_type=jnp.float32)
      + bfc_ref[...], 0.0)
  o_ref[...] = jnp.dot(
      z, wout_ref[...], preferred_element_type=jnp.float32) + bout_ref[...]


def _tc3_call(p, g, dinv, b, batch2d, wfc, bfc, wout, bout):
  return pl.pallas_call(
      _tc3_body,
      out_shape=jax.ShapeDtypeStruct((G, 1), jnp.float32),
  )(p, g, dinv, b, batch2d, wfc, bfc, wout, bout)


def kernel(x, edge_index, batch, W1, b1, W2, b2, Wfc1, bfc1, Wout, bout):
  pad = jnp.full((EPAD - E,), N, jnp.int32)
  srcp = jnp.concatenate([edge_index[0].astype(jnp.int32), pad]).reshape(
      NC, NS, NCH, CH)
  dstp = jnp.concatenate([edge_index[1].astype(jnp.int32), pad]).reshape(
      NC, NS, NCH, CH)
  x_pad = jnp.pad(x, ((0, NP - N), (0, 0)))
  batch2d = jnp.pad(
      batch.astype(jnp.int32), (0, NP - N), constant_values=G).reshape(NP, 1)

  hist = _hist_call()(dstp)
  d0 = hist[0, :, 0:1]
  d1 = hist[1, :, 0:1]

  g1, dinv = _tc1_call(x_pad, W1, d0, d1)
  p1 = _agg_call()(srcp, dstp, g1)
  g2 = _tc2_call(p1, g1, dinv, b1.reshape(1, H), W2)
  p2 = _agg_call()(srcp, dstp, g2)
  out = _tc3_call(p2, g2, dinv, b2.reshape(1, H), batch2d, Wfc1,
                  bfc1.reshape(1, G), Wout, bout.reshape(1, 1))
  return out


# speedup vs baseline: 35.1269x; 1.7614x over previous
import functools

import jax
import jax.numpy as jnp
from jax import lax
from jax.experimental import pallas as pl
from jax.experimental.pallas import tpu as pltpu
from jax.experimental.pallas import tpu_sc as plsc

N = 10000
E = 320000
D = 128
H = 64
G = 64

NC = 2
NS = 16
NW = NC * NS
CH = 128
NCH = 80
EPT = CH * NCH
EPAD = EPT * NW
NP = 10240
RPT = NP // NS
HW = 16

@functools.cache
def _sc_mesh():
  return plsc.VectorSubcoreMesh(
      core_axis_name="c", subcore_axis_name="s", num_cores=NC, num_subcores=NS)


def _hist_body(dst_hbm, out_hbm, acc, onesv, idxv, bufv):
  c = lax.axis_index("c")
  s = lax.axis_index("s")

  def fill(i, carry):
    onesv[i, :] = jnp.full((HW,), 1.0, jnp.float32)
    bufv[i, :] = jnp.zeros((HW,), jnp.float32)
    return carry

  lax.fori_loop(0, CH, fill, 0)
  for t in range(RPT // CH):
    pltpu.sync_copy(bufv, acc.at[pl.ds(s * RPT + t * CH, CH)])
  plsc.subcore_barrier()
  pltpu.sync_copy(dst_hbm.at[c, s], idxv)

  def body(j, carry):
    pltpu.sync_copy(onesv, acc.at[idxv.at[j]], add=True)
    return carry

  lax.fori_loop(0, NCH, body, 0)
  plsc.subcore_barrier()
  for t in range(RPT // CH):
    pltpu.sync_copy(acc.at[pl.ds(s * RPT + t * CH, CH)], bufv)
    pltpu.sync_copy(bufv, out_hbm.at[c, pl.ds(s * RPT + t * CH, CH)])


@functools.cache
def _hist_call():
  return pl.kernel(
      _hist_body,
      out_type=jax.ShapeDtypeStruct((NC, NP, HW), jnp.float32),
      mesh=_sc_mesh(),
      compiler_params=pltpu.CompilerParams(use_tc_tiling_on_sc=False),
      scratch_types=[
          pltpu.VMEM_SHARED((NP, HW), jnp.float32),
          pltpu.VMEM((CH, HW), jnp.float32),
          pltpu.VMEM((NCH, CH), jnp.int32),
          pltpu.VMEM((CH, HW), jnp.float32),
      ],
  )


NBUF = 2


def _agg_body(src_hbm, dst_hbm, g_hbm, out_hbm, acc, gbuf, srcv, dstv, rows,
              gsems, ssems, stsem):
  c = lax.axis_index("c")
  s = lax.axis_index("s")
  rowsa = rows[0]

  stage = pltpu.async_copy(
      g_hbm.at[pl.ds(s * RPT, RPT)], gbuf.at[pl.ds(s * RPT, RPT)], stsem)
  pltpu.sync_copy(src_hbm.at[c, s], srcv)
  pltpu.sync_copy(dst_hbm.at[c, s], dstv)

  def fill(i, carry):
    for q in range(H // 16):
      rowsa[i, pl.ds(q * 16, 16)] = jnp.zeros((16,), jnp.float32)
    return carry

  lax.fori_loop(0, CH, fill, 0)
  for t in range(RPT // CH):
    pltpu.sync_copy(rowsa, acc.at[pl.ds(s * RPT + t * CH, CH)])
  stage.wait()
  plsc.subcore_barrier()
  for b in range(NBUF):
    pltpu.async_copy(gbuf.at[srcv.at[b]], rows[b], gsems[b])

  def body(t, carry):
    j0 = NBUF * t
    for b in range(NBUF):
      j = j0 + b
      pltpu.make_async_copy(gbuf.at[srcv.at[0]], rows[b], gsems[b]).wait()
      pltpu.async_copy(rows[b], acc.at[dstv.at[j]], ssems[b], add=True)

      @pl.when(j + NBUF < NCH)
      def _():
        pltpu.make_async_copy(rows[b], acc.at[dstv.at[0]], ssems[b]).wait()
        pltpu.async_copy(gbuf.at[srcv.at[j + NBUF]], rows[b], gsems[b])

    return carry

  lax.fori_loop(0, NCH // NBUF, body, 0)
  for b in range(NBUF):
    pltpu.make_async_copy(rows[b], acc.at[dstv.at[0]], ssems[b]).wait()
  plsc.subcore_barrier()
  for t in range(RPT // CH):
    pltpu.sync_copy(acc.at[pl.ds(s * RPT + t * CH, CH)], rowsa)
    pltpu.sync_copy(rowsa, out_hbm.at[c, pl.ds(s * RPT + t * CH, CH)])


@functools.cache
def _agg_call():
  return pl.kernel(
      _agg_body,
      out_type=jax.ShapeDtypeStruct((NC, NP, H), jnp.float32),
      mesh=_sc_mesh(),
      compiler_params=pltpu.CompilerParams(use_tc_tiling_on_sc=False),
      scratch_types=[
          pltpu.VMEM_SHARED((NP, H), jnp.float32),
          pltpu.VMEM_SHARED((NP, H), jnp.float32),
          pltpu.VMEM((NCH, CH), jnp.int32),
          pltpu.VMEM((NCH, CH), jnp.int32),
          [pltpu.VMEM((CH, H), jnp.float32) for _ in range(NBUF)],
          [pltpu.SemaphoreType.DMA for _ in range(NBUF)],
          [pltpu.SemaphoreType.DMA for _ in range(NBUF)],
          pltpu.SemaphoreType.DMA,
      ],
  )


def _tc1_body(x_ref, w_ref, d0_ref, d1_ref, g_ref, dinv_ref):
  deg = d0_ref[...] + d1_ref[...] + 1.0
  dinv = lax.rsqrt(jnp.maximum(deg, 1.0))
  dinv_ref[...] = dinv
  g_ref[...] = dinv * jnp.dot(
      x_ref[...], w_ref[...], preferred_element_type=jnp.float32)


def _tc1_call(x, w, d0, d1):
  return pl.pallas_call(
      _tc1_body,
      out_shape=(
          jax.ShapeDtypeStruct((NP, H), jnp.float32),
          jax.ShapeDtypeStruct((NP, 1), jnp.float32),
      ),
  )(x, w, d0, d1)


def _tc2_body(p_ref, g_ref, dinv_ref, b_ref, w_ref, o_ref):
  es = p_ref[0] + p_ref[1] + g_ref[...]
  h = jnp.maximum(dinv_ref[...] * es + b_ref[...], 0.0)
  o_ref[...] = dinv_ref[...] * jnp.dot(
      h, w_ref[...], preferred_element_type=jnp.float32)


def _tc2_call(p, g, dinv, b, w):
  return pl.pallas_call(
      _tc2_body,
      out_shape=jax.ShapeDtypeStruct((NP, H), jnp.float32),
  )(p, g, dinv, b, w)


def _tc3_body(p_ref, g_ref, dinv_ref, b_ref, batch_ref, wfc_ref, bfc_ref,
              wout_ref, bout_ref, o_ref):
  h = jnp.maximum(
      dinv_ref[...] * (p_ref[0] + p_ref[1] + g_ref[...]) + b_ref[...], 0.0)
  seg = lax.broadcasted_iota(jnp.int32, (NP, G), 1)
  oh = (batch_ref[...] == seg).astype(jnp.float32)
  sums = lax.dot_general(
      oh, h, (((0,), (0,)), ((), ())), preferred_element_type=jnp.float32)
  counts = lax.dot_general(
      oh, jnp.ones((NP, 1), jnp.float32), (((0,), (0,)), ((), ())),
      preferred_element_type=jnp.float32)
  pooled = sums / jnp.maximum(counts, 1.0)
  z = jnp.maximum(
      jnp.dot(pooled, wfc_ref[...], preferred_element_type=jnp.float32)
      + bfc_ref[...], 0.0)
  o_ref[...] = jnp.dot(
      z, wout_ref[...], preferred_element_type=jnp.float32) + bout_ref[...]


def _tc3_call(p, g, dinv, b, batch2d, wfc, bfc, wout, bout):
  return pl.pallas_call(
      _tc3_body,
      out_shape=jax.ShapeDtypeStruct((G, 1), jnp.float32),
  )(p, g, dinv, b, batch2d, wfc, bfc, wout, bout)


def kernel(x, edge_index, batch, W1, b1, W2, b2, Wfc1, bfc1, Wout, bout):
  pad = jnp.full((EPAD - E,), N, jnp.int32)
  srcp = jnp.concatenate([edge_index[0].astype(jnp.int32), pad]).reshape(
      NC, NS, NCH, CH)
  dstp = jnp.concatenate([edge_index[1].astype(jnp.int32), pad]).reshape(
      NC, NS, NCH, CH)
  x_pad = jnp.pad(x, ((0, NP - N), (0, 0)))
  batch2d = jnp.pad(
      batch.astype(jnp.int32), (0, NP - N), constant_values=G).reshape(NP, 1)

  hist = _hist_call()(dstp)
  d0 = hist[0, :, 0:1]
  d1 = hist[1, :, 0:1]

  g1, dinv = _tc1_call(x_pad, W1, d0, d1)
  p1 = _agg_call()(srcp, dstp, g1)
  g2 = _tc2_call(p1, g1, dinv, b1.reshape(1, H), W2)
  p2 = _agg_call()(srcp, dstp, g2)
  out = _tc3_call(p2, g2, dinv, b2.reshape(1, H), batch2d, Wfc1,
                  bfc1.reshape(1, G), Wout, bout.reshape(1, 1))
  return out
